# Initial kernel scaffold; baseline (speedup 1.0000x reference)
import jax
import jax.numpy as jnp
from jax.experimental import pallas as pl


def kernel(classfy_out, index, confidence):
    raise NotImplementedError("write your pallas kernel here")


# speedup vs baseline: 1.0933x; 1.0933x over previous
import functools

import jax
import jax.numpy as jnp
from jax import lax
from jax.experimental import pallas as pl
from jax.experimental.pallas import tpu as pltpu
from jax.experimental.pallas import tpu_sc as plsc

B = 16384
C = 128

_info = plsc.get_sparse_core_info()
_NC, _NS = _info.num_cores, _info.num_subcores
NW = _NC * _NS
B_PER_W = B // NW
CHUNK = 128
N_CHUNK = B_PER_W // CHUNK


def _sc_gather(idx3, conf):
    mesh = plsc.VectorSubcoreMesh(core_axis_name="c", subcore_axis_name="s")

    @functools.partial(
        pl.kernel,
        mesh=mesh,
        out_type=jax.ShapeDtypeStruct((NW, N_CHUNK, CHUNK, C), jnp.float32),
        scratch_types=[
            pltpu.VMEM((N_CHUNK, CHUNK), jnp.int32),
            pltpu.VMEM((N_CHUNK, CHUNK, C), jnp.float32),
            pltpu.SemaphoreType.DMA,
        ],
    )
    def k(idx_hbm, conf_hbm, out_hbm, idx_v, rows_v, sem):
        wid = lax.axis_index("s") * _NC + lax.axis_index("c")
        pltpu.sync_copy(idx_hbm.at[wid], idx_v)
        copies = [
            pltpu.async_copy(conf_hbm.at[idx_v.at[j]], rows_v.at[j], sem)
            for j in range(N_CHUNK)
        ]
        for cp in copies:
            cp.wait()
        pltpu.sync_copy(rows_v, out_hbm.at[wid])

    return k(idx3, conf)


def _tc_loss(pred, target):
    BLK = 2048
    grid = B // BLK

    def body(p_ref, t_ref, o_ref, acc_ref):
        @pl.when(pl.program_id(0) == 0)
        def _():
            acc_ref[0, 0] = 0.0

        acc_ref[0, 0] += jnp.sum(jnp.log(p_ref[...]) * t_ref[...])

        @pl.when(pl.program_id(0) == grid - 1)
        def _():
            o_ref[0, 0] = -acc_ref[0, 0] * (1.0 / B)

    out = pl.pallas_call(
        body,
        grid=(grid,),
        in_specs=[
            pl.BlockSpec((BLK, C), lambda i: (i, 0)),
            pl.BlockSpec((BLK, C), lambda i: (i, 0)),
        ],
        out_specs=pl.BlockSpec(memory_space=pltpu.SMEM),
        out_shape=jax.ShapeDtypeStruct((1, 1), jnp.float32),
        scratch_shapes=[pltpu.SMEM((1, 1), jnp.float32)],
    )(pred, target)
    return out[0, 0]


def kernel(classfy_out, index, confidence):
    idx3 = index.reshape(NW, N_CHUNK, CHUNK)
    target = _sc_gather(idx3, confidence)
    return _tc_loss(classfy_out, target.reshape(B, C))
